# baseline (speedup 1.0000x reference)
import functools

import jax
import jax.numpy as jnp
from jax import lax
from jax.experimental import pallas as pl
from jax.experimental.pallas import tpu as pltpu
from jax.experimental.pallas import tpu_sc as plsc

_MAX_VAL = 16383
_S = 16384
_D = 4096
_L = 16
_NW = 32
_EPW = _S // _NW
_VPW = _EPW // _L
_ROWS = 1024

_sc_mesh = plsc.VectorSubcoreMesh(core_axis_name="c", subcore_axis_name="s",
                                  num_cores=2)


@functools.partial(
    pl.kernel,
    out_type=jax.ShapeDtypeStruct((_S,), jnp.int32),
    mesh=_sc_mesh,
    scratch_types=[
        pltpu.VMEM((_S,), jnp.int32),
        pltpu.VMEM((_EPW,), jnp.int32),
    ],
    compiler_params=pltpu.CompilerParams(needs_layout_passes=False),
)
def _sc_index_kernel(mask_hbm, out_hbm, mask_v, out_v):
    wid = lax.axis_index("s") * 2 + lax.axis_index("c")
    base = wid * _EPW
    pltpu.sync_copy(mask_hbm, mask_v)

    def _acc(j, acc):
        return acc + mask_v[pl.ds(j * _L, _L)]
    acc = lax.fori_loop(0, wid * _VPW, _acc, jnp.zeros((_L,), jnp.int32))
    off = jnp.sum(acc) - 1

    for j in range(_VPW):
        v = mask_v[pl.ds(base + j * _L, _L)]
        cs = jnp.cumsum(v) + off
        out_v[pl.ds(j * _L, _L)] = jnp.clip(cs, 0, _MAX_VAL)
        off = off + jnp.sum(v)

    pltpu.sync_copy(out_v, out_hbm.at[pl.ds(base, _EPW)])


def _bcast_kernel(idx_ref, out_ref):
    out_ref[...] = jnp.broadcast_to(idx_ref[...], (_ROWS, _D))


@jax.jit
def kernel(mask_1d, inputs_embeds_row):
    del inputs_embeds_row
    idx1d = _sc_index_kernel(mask_1d.astype(jnp.int32))
    idx_col = idx1d.reshape(_S, 1)
    return pl.pallas_call(
        _bcast_kernel,
        grid=(_S // _ROWS,),
        in_specs=[pl.BlockSpec((_ROWS, 1), lambda i: (i, 0))],
        out_specs=pl.BlockSpec((_ROWS, _D), lambda i: (i, 0)),
        out_shape=jax.ShapeDtypeStruct((_S, _D), jnp.int32),
    )(idx_col)


# speedup vs baseline: 1.0693x; 1.0693x over previous
import functools

import jax
import jax.numpy as jnp
from jax import lax
from jax.experimental import pallas as pl
from jax.experimental.pallas import tpu as pltpu
from jax.experimental.pallas import tpu_sc as plsc

_MAX_VAL = 16383
_S = 16384
_D = 4096
_L = 16
_NW = 32
_EPW = _S // _NW
_VPW = _EPW // _L
_CHUNK = 128
_ROWS = 512
_COLS = _ROWS // _CHUNK

_sc_mesh = plsc.VectorSubcoreMesh(core_axis_name="c", subcore_axis_name="s",
                                  num_cores=2)


@functools.partial(
    pl.kernel,
    out_type=jax.ShapeDtypeStruct((_S,), jnp.int32),
    mesh=_sc_mesh,
    scratch_types=[
        pltpu.VMEM((_S,), jnp.int32),
        pltpu.VMEM((_EPW,), jnp.int32),
    ],
    compiler_params=pltpu.CompilerParams(needs_layout_passes=False),
)
def _sc_index_kernel(mask_hbm, out_hbm, mask_v, out_v):
    wid = lax.axis_index("s") * 2 + lax.axis_index("c")
    base = wid * _EPW
    pltpu.sync_copy(mask_hbm, mask_v)

    def _acc(j, acc):
        return acc + mask_v[pl.ds(j * _L, _L)]
    acc = lax.fori_loop(0, wid * _VPW, _acc, jnp.zeros((_L,), jnp.int32))
    off = jnp.sum(acc) - 1

    for j in range(_VPW):
        v = mask_v[pl.ds(base + j * _L, _L)]
        cs = jnp.cumsum(v) + off
        out_v[pl.ds(j * _L, _L)] = jnp.clip(cs, 0, _MAX_VAL)
        off = off + jnp.sum(v)

    pltpu.sync_copy(out_v, out_hbm.at[pl.ds(base, _EPW)])


def _bcast_kernel(idx2d_ref, out_ref, hi_ref, lo_ref):
    i = pl.program_id(0)

    @pl.when(i == 0)
    def _prep():
        idx = idx2d_ref[...].astype(jnp.float32)
        hi = jnp.floor(idx * (1.0 / _CHUNK))
        lo = idx - hi * float(_CHUNK)
        hi_ref[...] = hi.T
        lo_ref[...] = lo.T

    sub = jax.lax.broadcasted_iota(jnp.int32, (_CHUNK, 1), 0)
    for j in range(_COLS):
        onehot = (sub == i * _COLS + j).astype(jnp.float32)
        hi_col = jnp.dot(hi_ref[...], onehot,
                         preferred_element_type=jnp.float32)
        lo_col = jnp.dot(lo_ref[...], onehot,
                         preferred_element_type=jnp.float32)
        colv = hi_col * float(_CHUNK) + lo_col
        out_ref[pl.ds(j * _CHUNK, _CHUNK), :] = jnp.broadcast_to(
            colv.astype(jnp.int32), (_CHUNK, _D))


@jax.jit
def kernel(mask_1d, inputs_embeds_row):
    del inputs_embeds_row
    idx1d = _sc_index_kernel(mask_1d.astype(jnp.int32))
    idx2d = idx1d.reshape(_S // _CHUNK, _CHUNK)
    return pl.pallas_call(
        _bcast_kernel,
        grid=(_S // _ROWS,),
        in_specs=[pl.BlockSpec((_CHUNK, _CHUNK), lambda i: (0, 0))],
        out_specs=pl.BlockSpec((_ROWS, _D), lambda i: (i, 0)),
        out_shape=jax.ShapeDtypeStruct((_S, _D), jnp.int32),
        scratch_shapes=[pltpu.VMEM((_CHUNK, _CHUNK), jnp.float32),
                        pltpu.VMEM((_CHUNK, _CHUNK), jnp.float32)],
    )(idx2d)


# speedup vs baseline: 1.1266x; 1.0536x over previous
import functools

import jax
import jax.numpy as jnp
from jax import lax
from jax.experimental import pallas as pl
from jax.experimental.pallas import tpu as pltpu
from jax.experimental.pallas import tpu_sc as plsc

_MAX_VAL = 16383
_S = 16384
_D = 4096
_L = 16
_NW = 32
_EPW = _S // _NW
_VPW = _EPW // _L
_CHUNK = 128
_ROWS = 512
_COLS = _ROWS // _CHUNK

_sc_mesh = plsc.VectorSubcoreMesh(core_axis_name="c", subcore_axis_name="s",
                                  num_cores=2)


@functools.partial(
    pl.kernel,
    out_type=jax.ShapeDtypeStruct((_S // _CHUNK, _CHUNK), jnp.int32),
    mesh=_sc_mesh,
    scratch_types=[
        pltpu.VMEM((_EPW,), jnp.int32),
        pltpu.VMEM((_EPW,), jnp.int32),
        pltpu.VMEM((_NW * _L,), jnp.int32),
        pltpu.VMEM_SHARED((_NW * _L,), jnp.int32),
        pltpu.VMEM((_EPW,), jnp.int32),
    ],
    compiler_params=pltpu.CompilerParams(needs_layout_passes=False),
)
def _sc_index_kernel(mask_hbm, out_hbm, own_v, mir_v, tot_v, board, out_v):
    c = lax.axis_index("c")
    s = lax.axis_index("s")
    wid = c * 16 + s
    mir = (1 - c) * 16 + s
    base = wid * _EPW

    pltpu.sync_copy(mask_hbm.at[pl.ds(base, _EPW)], own_v)
    pltpu.sync_copy(mask_hbm.at[pl.ds(mir * _EPW, _EPW)], mir_v)

    own_acc = jnp.zeros((_L,), jnp.int32)
    mir_acc = jnp.zeros((_L,), jnp.int32)
    for j in range(_VPW):
        own_acc = own_acc + own_v[pl.ds(j * _L, _L)]
        mir_acc = mir_acc + mir_v[pl.ds(j * _L, _L)]
    tot_v[pl.ds(wid * _L, _L)] = jnp.full((_L,), jnp.sum(own_acc), jnp.int32)
    tot_v[pl.ds(mir * _L, _L)] = jnp.full((_L,), jnp.sum(mir_acc), jnp.int32)

    pltpu.sync_copy(tot_v.at[pl.ds(wid * _L, _L)], board.at[pl.ds(wid * _L, _L)])
    pltpu.sync_copy(tot_v.at[pl.ds(mir * _L, _L)], board.at[pl.ds(mir * _L, _L)])
    plsc.subcore_barrier()
    pltpu.sync_copy(board, tot_v)

    pre = jnp.zeros((_L,), jnp.int32)
    for j in range(_NW):
        row = tot_v[pl.ds(j * _L, _L)]
        pre = pre + jnp.where(j < wid, row, jnp.zeros((_L,), jnp.int32))
    off = jnp.max(pre) - 1

    for j in range(_VPW):
        v = own_v[pl.ds(j * _L, _L)]
        cs = jnp.cumsum(v) + off
        out_v[pl.ds(j * _L, _L)] = jnp.clip(cs, 0, _MAX_VAL)
        off = off + jnp.sum(v)

    for j in range(_EPW // _CHUNK):
        pltpu.sync_copy(out_v.at[pl.ds(j * _CHUNK, _CHUNK)],
                        out_hbm.at[4 * wid + j])


def _bcast_kernel(idx2d_ref, out_ref, hi_ref, lo_ref):
    i = pl.program_id(0)

    @pl.when(i == 0)
    def _prep():
        idx = idx2d_ref[...].astype(jnp.float32)
        hi = jnp.floor(idx * (1.0 / _CHUNK))
        lo = idx - hi * float(_CHUNK)
        hi_ref[...] = hi.T
        lo_ref[...] = lo.T

    sub = jax.lax.broadcasted_iota(jnp.int32, (_CHUNK, 1), 0)
    for j in range(_COLS):
        onehot = (sub == i * _COLS + j).astype(jnp.float32)
        hi_col = jnp.dot(hi_ref[...], onehot,
                         preferred_element_type=jnp.float32)
        lo_col = jnp.dot(lo_ref[...], onehot,
                         preferred_element_type=jnp.float32)
        colv = hi_col * float(_CHUNK) + lo_col
        out_ref[pl.ds(j * _CHUNK, _CHUNK), :] = jnp.broadcast_to(
            colv.astype(jnp.int32), (_CHUNK, _D))


@jax.jit
def kernel(mask_1d, inputs_embeds_row):
    del inputs_embeds_row
    idx2d = _sc_index_kernel(mask_1d.astype(jnp.int32))
    return pl.pallas_call(
        _bcast_kernel,
        grid=(_S // _ROWS,),
        in_specs=[pl.BlockSpec((_CHUNK, _CHUNK), lambda i: (0, 0))],
        out_specs=pl.BlockSpec((_ROWS, _D), lambda i: (i, 0)),
        out_shape=jax.ShapeDtypeStruct((_S, _D), jnp.int32),
        scratch_shapes=[pltpu.VMEM((_CHUNK, _CHUNK), jnp.float32),
                        pltpu.VMEM((_CHUNK, _CHUNK), jnp.float32)],
    )(idx2d)


# speedup vs baseline: 1.1335x; 1.0062x over previous
import functools

import jax
import jax.numpy as jnp
from jax import lax
from jax.experimental import pallas as pl
from jax.experimental.pallas import tpu as pltpu
from jax.experimental.pallas import tpu_sc as plsc

_MAX_VAL = 16383
_S = 16384
_D = 4096
_L = 16
_NW = 32
_EPW = _S // _NW
_VPW = _EPW // _L
_CHUNK = 128
_ROWS = 256
_COLS = _ROWS // _CHUNK

_sc_mesh = plsc.VectorSubcoreMesh(core_axis_name="c", subcore_axis_name="s",
                                  num_cores=2)


@functools.partial(
    pl.kernel,
    out_type=jax.ShapeDtypeStruct((_S // _CHUNK, _CHUNK), jnp.int32),
    mesh=_sc_mesh,
    scratch_types=[
        pltpu.VMEM((_EPW,), jnp.int32),
        pltpu.VMEM((_EPW,), jnp.int32),
        pltpu.VMEM((_NW * _L,), jnp.int32),
        pltpu.VMEM_SHARED((_NW * _L,), jnp.int32),
        pltpu.VMEM((_EPW,), jnp.int32),
    ],
    compiler_params=pltpu.CompilerParams(needs_layout_passes=False),
)
def _sc_index_kernel(mask_hbm, out_hbm, own_v, mir_v, tot_v, board, out_v):
    c = lax.axis_index("c")
    s = lax.axis_index("s")
    wid = c * 16 + s
    mir = (1 - c) * 16 + s
    base = wid * _EPW

    pltpu.sync_copy(mask_hbm.at[pl.ds(base, _EPW)], own_v)
    pltpu.sync_copy(mask_hbm.at[pl.ds(mir * _EPW, _EPW)], mir_v)

    own_acc = jnp.zeros((_L,), jnp.int32)
    mir_acc = jnp.zeros((_L,), jnp.int32)
    for j in range(_VPW):
        own_acc = own_acc + own_v[pl.ds(j * _L, _L)]
        mir_acc = mir_acc + mir_v[pl.ds(j * _L, _L)]
    tot_v[pl.ds(wid * _L, _L)] = jnp.full((_L,), jnp.sum(own_acc), jnp.int32)
    tot_v[pl.ds(mir * _L, _L)] = jnp.full((_L,), jnp.sum(mir_acc), jnp.int32)

    pltpu.sync_copy(tot_v.at[pl.ds(wid * _L, _L)], board.at[pl.ds(wid * _L, _L)])
    pltpu.sync_copy(tot_v.at[pl.ds(mir * _L, _L)], board.at[pl.ds(mir * _L, _L)])
    plsc.subcore_barrier()
    pltpu.sync_copy(board, tot_v)

    pre = jnp.zeros((_L,), jnp.int32)
    for j in range(_NW):
        row = tot_v[pl.ds(j * _L, _L)]
        pre = pre + jnp.where(j < wid, row, jnp.zeros((_L,), jnp.int32))
    off = jnp.max(pre) - 1

    for j in range(_VPW):
        v = own_v[pl.ds(j * _L, _L)]
        cs = jnp.cumsum(v) + off
        out_v[pl.ds(j * _L, _L)] = jnp.clip(cs, 0, _MAX_VAL)
        off = off + jnp.sum(v)

    for j in range(_EPW // _CHUNK):
        pltpu.sync_copy(out_v.at[pl.ds(j * _CHUNK, _CHUNK)],
                        out_hbm.at[4 * wid + j])


def _bcast_kernel(idx2d_ref, out_ref, hi_ref, lo_ref):
    i = pl.program_id(0)

    @pl.when(i == 0)
    def _prep():
        idx = idx2d_ref[...].astype(jnp.float32)
        hi = jnp.floor(idx * (1.0 / _CHUNK))
        lo = idx - hi * float(_CHUNK)
        hi_ref[...] = hi.T
        lo_ref[...] = lo.T

    sub = jax.lax.broadcasted_iota(jnp.int32, (_CHUNK, 1), 0)
    for j in range(_COLS):
        onehot = (sub == i * _COLS + j).astype(jnp.float32)
        hi_col = jnp.dot(hi_ref[...], onehot,
                         preferred_element_type=jnp.float32)
        lo_col = jnp.dot(lo_ref[...], onehot,
                         preferred_element_type=jnp.float32)
        colv = hi_col * float(_CHUNK) + lo_col
        out_ref[pl.ds(j * _CHUNK, _CHUNK), :] = jnp.broadcast_to(
            colv.astype(jnp.int32), (_CHUNK, _D))


@jax.jit
def kernel(mask_1d, inputs_embeds_row):
    del inputs_embeds_row
    idx2d = _sc_index_kernel(mask_1d.astype(jnp.int32))
    return pl.pallas_call(
        _bcast_kernel,
        grid=(_S // _ROWS,),
        in_specs=[pl.BlockSpec((_CHUNK, _CHUNK), lambda i: (0, 0))],
        out_specs=pl.BlockSpec((_ROWS, _D), lambda i: (i, 0)),
        out_shape=jax.ShapeDtypeStruct((_S, _D), jnp.int32),
        scratch_shapes=[pltpu.VMEM((_CHUNK, _CHUNK), jnp.float32),
                        pltpu.VMEM((_CHUNK, _CHUNK), jnp.float32)],
    )(idx2d)


# speedup vs baseline: 1.3749x; 1.2130x over previous
import functools

import jax
import jax.numpy as jnp
from jax.experimental import pallas as pl
from jax.experimental.pallas import tpu as pltpu

_MAX_VAL = 16383
_S = 16384
_D = 4096
_CHUNK = 128
_ROWS = 256
_COLS = _ROWS // _CHUNK


def _idx_kernel(maskT_ref, out_ref):
    m = maskT_ref[...].astype(jnp.float32)
    row = jax.lax.broadcasted_iota(jnp.int32, (_CHUNK, _CHUNK), 0)
    col = jax.lax.broadcasted_iota(jnp.int32, (_CHUNK, _CHUNK), 1)
    l_incl = (col <= row).astype(jnp.float32)
    u_strict = (row < col).astype(jnp.float32)
    csT = jnp.dot(l_incl, m, preferred_element_type=jnp.float32)
    prefT = jnp.dot(csT[_CHUNK - 1:_CHUNK, :], u_strict,
                    preferred_element_type=jnp.float32)
    idxT = jnp.clip(csT + prefT - 1.0, 0.0, float(_MAX_VAL))
    out_ref[...] = idxT.astype(jnp.int32)


def _bcast_kernel(idxT_ref, out_ref, hi_ref, lo_ref):
    i = pl.program_id(0)

    @pl.when(i == 0)
    def _prep():
        idx = idxT_ref[...].astype(jnp.float32)
        hi = jnp.floor(idx * (1.0 / _CHUNK))
        hi_ref[...] = hi
        lo_ref[...] = idx - hi * float(_CHUNK)

    sub = jax.lax.broadcasted_iota(jnp.int32, (_CHUNK, 1), 0)
    for j in range(_COLS):
        onehot = (sub == i * _COLS + j).astype(jnp.float32)
        hi_col = jnp.dot(hi_ref[...], onehot,
                         preferred_element_type=jnp.float32)
        lo_col = jnp.dot(lo_ref[...], onehot,
                         preferred_element_type=jnp.float32)
        colv = hi_col * float(_CHUNK) + lo_col
        out_ref[pl.ds(j * _CHUNK, _CHUNK), :] = jnp.broadcast_to(
            colv.astype(jnp.int32), (_CHUNK, _D))


@jax.jit
def kernel(mask_1d, inputs_embeds_row):
    del inputs_embeds_row
    maskT = mask_1d.astype(jnp.int32).reshape(_S // _CHUNK, _CHUNK).T
    idxT = pl.pallas_call(
        _idx_kernel,
        out_shape=jax.ShapeDtypeStruct((_CHUNK, _CHUNK), jnp.int32),
    )(maskT)
    return pl.pallas_call(
        _bcast_kernel,
        grid=(_S // _ROWS,),
        in_specs=[pl.BlockSpec((_CHUNK, _CHUNK), lambda i: (0, 0))],
        out_specs=pl.BlockSpec((_ROWS, _D), lambda i: (i, 0)),
        out_shape=jax.ShapeDtypeStruct((_S, _D), jnp.int32),
        scratch_shapes=[pltpu.VMEM((_CHUNK, _CHUNK), jnp.float32),
                        pltpu.VMEM((_CHUNK, _CHUNK), jnp.float32)],
    )(idxT)
